# Initial kernel scaffold; baseline (speedup 1.0000x reference)
import jax
import jax.numpy as jnp
from jax.experimental import pallas as pl


def kernel(tgt, memory, sa_wqkv, sa_bqkv, sa_wo, sa_bo, ma_wqkv, ma_bqkv, ma_wo, ma_bo, router_w, router_b, noise_w, noise_b, w1, b1, w2, b2, ln1_g, ln1_b, ln2_g, ln2_b, ln3_g, ln3_b):
    raise NotImplementedError("write your pallas kernel here")


# speedup vs baseline: 1.0686x; 1.0686x over previous
import functools

import jax
import jax.numpy as jnp
from jax.experimental import pallas as pl
from jax.experimental.pallas import tpu as pltpu

S, D, H, E, K, FFN = 2048, 768, 12, 8, 2, 2048
DH = D // H

_bf = jnp.bfloat16


def _dotT(a, b):
    return jax.lax.dot_general(
        a.astype(_bf), b.astype(_bf), (((1,), (1,)), ((), ())),
        preferred_element_type=jnp.float32)


def _ln_rows(x, g, b):
    m = jnp.mean(x, axis=-1, keepdims=True)
    v = jnp.mean((x - m) ** 2, axis=-1, keepdims=True)
    return (x - m) / jnp.sqrt(v + 1e-5) * g + b


def _qkv_body(xq_ref, xkv_ref, wq_ref, wk_ref, wv_ref, bq_ref, bk_ref, bv_ref,
              q_ref, k_ref, v_ref):
    q_ref[...] = _dotT(xq_ref[...], wq_ref[...]) + bq_ref[...]
    k_ref[...] = _dotT(xkv_ref[...], wk_ref[...]) + bk_ref[...]
    v_ref[...] = _dotT(xkv_ref[...], wv_ref[...]) + bv_ref[...]


def _qkv_proj(xq, xkv, wqkv, bqkv):
    RB = 256
    grid = (S // RB,)
    wq, wk, wv = wqkv[:D], wqkv[D:2 * D], wqkv[2 * D:]
    bq, bk, bv = bqkv[:D], bqkv[D:2 * D], bqkv[2 * D:]
    full_w = pl.BlockSpec((D, D), lambda i: (0, 0))
    full_b = pl.BlockSpec((D,), lambda i: (0,))
    row = pl.BlockSpec((RB, D), lambda i: (i, 0))
    return pl.pallas_call(
        _qkv_body,
        grid=grid,
        in_specs=[row, row, full_w, full_w, full_w, full_b, full_b, full_b],
        out_specs=[row, row, row],
        out_shape=[jax.ShapeDtypeStruct((S, D), jnp.float32)] * 3,
    )(xq, xkv, wq, wk, wv, bq, bk, bv)


def _attn_body(q_ref, k_ref, v_ref, o_ref):
    s = _dotT(q_ref[0], k_ref[0]) * 0.125
    m = jnp.max(s, axis=1, keepdims=True)
    p = jnp.exp(s - m)
    l = jnp.sum(p, axis=1, keepdims=True)
    a = p / l
    o_ref[0] = jax.lax.dot_general(
        a.astype(_bf), v_ref[0].astype(_bf), (((1,), (0,)), ((), ())),
        preferred_element_type=jnp.float32)


def _attention(q, k, v):
    QB = 1024
    grid = (H, S // QB)
    return pl.pallas_call(
        _attn_body,
        grid=grid,
        in_specs=[
            pl.BlockSpec((1, QB, DH), lambda h, qb: (h, qb, 0)),
            pl.BlockSpec((1, S, DH), lambda h, qb: (h, 0, 0)),
            pl.BlockSpec((1, S, DH), lambda h, qb: (h, 0, 0)),
        ],
        out_specs=pl.BlockSpec((1, QB, DH), lambda h, qb: (h, qb, 0)),
        out_shape=jax.ShapeDtypeStruct((H, S, DH), jnp.float32),
    )(q, k, v)


def _proj_ln_body(o_ref, res_ref, wo_ref, bo_ref, g_ref, b_ref, y_ref):
    y = _dotT(o_ref[...], wo_ref[...]) + bo_ref[...]
    x = res_ref[...] + y
    y_ref[...] = _ln_rows(x, g_ref[...], b_ref[...])


def _proj_ln(o, res, wo, bo, g, b):
    RB = 256
    row = pl.BlockSpec((RB, D), lambda i: (i, 0))
    return pl.pallas_call(
        _proj_ln_body,
        grid=(S // RB,),
        in_specs=[row, row,
                  pl.BlockSpec((D, D), lambda i: (0, 0)),
                  pl.BlockSpec((D,), lambda i: (0,)),
                  pl.BlockSpec((D,), lambda i: (0,)),
                  pl.BlockSpec((D,), lambda i: (0,))],
        out_specs=row,
        out_shape=jax.ShapeDtypeStruct((S, D), jnp.float32),
    )(o, res, wo, bo, g, b)


def _mha_block(xq, xkv, wqkv, bqkv, wo, bo, res, ln_g, ln_b):
    q, k, v = _qkv_proj(xq, xkv, wqkv, bqkv)
    qh = q.reshape(S, H, DH).transpose(1, 0, 2)
    kh = k.reshape(S, H, DH).transpose(1, 0, 2)
    vh = v.reshape(S, H, DH).transpose(1, 0, 2)
    oh = _attention(qh, kh, vh)
    o = oh.transpose(1, 0, 2).reshape(S, D)
    return _proj_ln(o, res, wo, bo, ln_g, ln_b)


def _router_body(x_ref, nw_ref, nb_ref, gate_ref):
    logits = _dotT(x_ref[...], nw_ref[...]) + nb_ref[...]
    iota = jax.lax.broadcasted_iota(jnp.int32, (S, E), 1)
    v1 = jnp.max(logits, axis=1, keepdims=True)
    i1 = jnp.min(jnp.where(logits == v1, iota, E), axis=1, keepdims=True)
    l2 = jnp.where(iota == i1, -jnp.inf, logits)
    v2 = jnp.max(l2, axis=1, keepdims=True)
    i2 = jnp.min(jnp.where(l2 == v2, iota, E), axis=1, keepdims=True)
    e2 = jnp.exp(v2 - v1)
    den = 1.0 + e2
    g1 = 1.0 / den
    g2 = e2 / den
    gate_ref[...] = jnp.where(iota == i1, g1, jnp.where(iota == i2, g2, 0.0))


def _router(x, noise_w, noise_b):
    return pl.pallas_call(
        _router_body,
        grid=(1,),
        in_specs=[pl.BlockSpec((S, D), lambda i: (0, 0)),
                  pl.BlockSpec((E, D), lambda i: (0, 0)),
                  pl.BlockSpec((E,), lambda i: (0,))],
        out_specs=pl.BlockSpec((S, E), lambda i: (0, 0)),
        out_shape=jax.ShapeDtypeStruct((S, E), jnp.float32),
    )(x, noise_w, noise_b)


def _moe_body(x_ref, gate_ref, w1_ref, b1_ref, w2_ref, b2_ref, g_ref, b_ref,
              y_ref, acc_ref):
    e = pl.program_id(1)
    h = jnp.maximum(_dotT(x_ref[...], w1_ref[0]) + b1_ref[0], 0.0)
    eo = _dotT(h, w2_ref[0]) + b2_ref[0]
    iota = jax.lax.broadcasted_iota(jnp.int32, gate_ref.shape, 1)
    g_col = jnp.sum(jnp.where(iota == e, gate_ref[...], 0.0), axis=1,
                    keepdims=True)
    contrib = eo * g_col

    @pl.when(e == 0)
    def _():
        acc_ref[...] = contrib

    @pl.when(e > 0)
    def _():
        acc_ref[...] += contrib

    @pl.when(e == E - 1)
    def _():
        y_ref[...] = _ln_rows(x_ref[...] + acc_ref[...], g_ref[...], b_ref[...])


def _moe_dense(x, gate, w1, b1, w2, b2, ln_g, ln_b):
    RB = 1024
    grid = (S // RB, E)
    return pl.pallas_call(
        _moe_body,
        grid=grid,
        in_specs=[
            pl.BlockSpec((RB, D), lambda rb, e: (rb, 0)),
            pl.BlockSpec((RB, E), lambda rb, e: (rb, 0)),
            pl.BlockSpec((1, FFN, D), lambda rb, e: (e, 0, 0)),
            pl.BlockSpec((1, 1, FFN), lambda rb, e: (e, 0, 0)),
            pl.BlockSpec((1, D, FFN), lambda rb, e: (e, 0, 0)),
            pl.BlockSpec((1, 1, D), lambda rb, e: (e, 0, 0)),
            pl.BlockSpec((D,), lambda rb, e: (0,)),
            pl.BlockSpec((D,), lambda rb, e: (0,)),
        ],
        out_specs=pl.BlockSpec((RB, D), lambda rb, e: (rb, 0)),
        out_shape=jax.ShapeDtypeStruct((S, D), jnp.float32),
        scratch_shapes=[pltpu.VMEM((RB, D), jnp.float32)],
    )(x, gate, w1, b1.reshape(E, 1, FFN), w2, b2.reshape(E, 1, D), ln_g, ln_b)


def kernel(tgt, memory, sa_wqkv, sa_bqkv, sa_wo, sa_bo, ma_wqkv, ma_bqkv,
           ma_wo, ma_bo, router_w, router_b, noise_w, noise_b, w1, b1, w2, b2,
           ln1_g, ln1_b, ln2_g, ln2_b, ln3_g, ln3_b):
    x0 = tgt.reshape(S, D)
    mem = memory.reshape(S, D)
    x1 = _mha_block(x0, x0, sa_wqkv, sa_bqkv, sa_wo, sa_bo, x0, ln1_g, ln1_b)
    x2 = _mha_block(x1, mem, ma_wqkv, ma_bqkv, ma_wo, ma_bo, x1, ln2_g, ln2_b)
    gate = _router(x2, noise_w, noise_b)
    y = _moe_dense(x2, gate, w1, b1, w2, b2, ln3_g, ln3_b)
    return y.reshape(S, 1, D)
